# Initial kernel scaffold; baseline (speedup 1.0000x reference)
import jax
import jax.numpy as jnp
from jax.experimental import pallas as pl


def kernel(feat, graph_turn, graph_control, graph_intersect, graph_idcs, graph_ctrs, actors, actor_idcs, actor_ctrs, params):
    raise NotImplementedError("write your pallas kernel here")


# speedup vs baseline: 2.0845x; 2.0845x over previous
import jax
import jax.numpy as jnp
from jax.experimental import pallas as pl

N_MAP = 50000
N_ACT = 64
N_D = 128
EPS = 1e-5
NB = 512
MPAD = ((N_MAP + NB - 1) // NB) * NB


def _ln(x, w, b):
    m = jnp.mean(x, axis=-1, keepdims=True)
    v = jnp.mean((x - m) ** 2, axis=-1, keepdims=True)
    return (x - m) / jnp.sqrt(v + EPS) * w + b


def _dot(a, b):
    return jnp.dot(a, b, preferred_element_type=jnp.float32)


def _pre_body(xin_ref, w_ref, gw_ref, gb_ref, o_ref):
    y = _dot(xin_ref[...], w_ref[...])
    o_ref[...] = jax.nn.relu(_ln(y, gw_ref[...], gb_ref[...]))


def _layer_pre_body(x_ref, ctrs_ref, qw_ref, qgw_ref, qgb_ref, wq_ref,
                    agtw_ref, d0w_ref, q2_ref, acc0_ref, g0_ref):
    x = x_ref[...]
    q = jax.nn.relu(_ln(_dot(x, qw_ref[...]), qgw_ref[...], qgb_ref[...]))
    q2_ref[...] = _dot(q, wq_ref[...])
    acc0_ref[...] = _dot(x, agtw_ref[...])
    c = ctrs_ref[...]
    d0w = d0w_ref[...]
    g0_ref[...] = c[:, 0:1] * d0w[0:1, :] + c[:, 1:2] * d0w[1:2, :]


def _small_body(actc_ref, d0w_ref, d0b_ref, actors_ref, wc_ref,
                a0b_ref, c2_ref):
    ac = actc_ref[...]
    d0w = d0w_ref[...]
    a0 = ac[:, 0:1] * d0w[0:1, :] + ac[:, 1:2] * d0w[1:2, :]
    a0b_ref[...] = d0b_ref[...] - a0
    c2_ref[...] = _dot(actors_ref[...], wc_ref[...])


def _att_body(q2_ref, g0_ref, acc0_ref, ctrs_ref, actc_ref, a0b_ref, c2_ref,
              d1w_ref, d1gw_ref, d1gb_ref, wd_ref, c0gw_ref, c0gb_ref,
              c1w_ref, out_ref):
    q2 = q2_ref[...]
    g0 = g0_ref[...]
    ctrs = ctrs_ref[...]
    d1w = d1w_ref[...]
    wd = wd_ref[...]
    c1w = c1w_ref[...]
    d1gw = d1gw_ref[...]
    d1gb = d1gb_ref[...]
    c0gw = c0gw_ref[...]
    c0gb = c0gb_ref[...]

    def body(j, acc):
        a0b = a0b_ref[pl.ds(j, 1), :]
        c2j = c2_ref[pl.ds(j, 1), :]
        aj = actc_ref[pl.ds(j, 1), :]
        d1 = jax.nn.relu(g0 + a0b)
        t = jax.nn.relu(_ln(_dot(d1, d1w), d1gw, d1gb))
        pre = _dot(t, wd) + q2 + c2j
        cc = jax.nn.relu(_ln(pre, c0gw, c0gb))
        o = _dot(cc, c1w)
        diff = ctrs - aj
        dist = jnp.sqrt(jnp.sum(diff * diff, axis=1, keepdims=True))
        return acc + jnp.where(dist <= 0.5, o, 0.0)

    out_ref[...] = jax.lax.fori_loop(0, N_ACT, body, acc0_ref[...])


def _post_body(acc_ref, res_ref, ngw_ref, ngb_ref, lw_ref, lgw_ref, lgb_ref,
               out_ref):
    a = jax.nn.relu(_ln(acc_ref[...], ngw_ref[...], ngb_ref[...]))
    y = _ln(_dot(a, lw_ref[...]), lgw_ref[...], lgb_ref[...])
    out_ref[...] = jax.nn.relu(y + res_ref[...])


def _row_spec(cols):
    return pl.BlockSpec((NB, cols), lambda i: (i, 0))


def _full_spec(shape):
    nd = len(shape)
    return pl.BlockSpec(shape, lambda *_: (0,) * nd)


def _row_call(body, ins, full_ins, n_out, out_cols=N_D):
    grid = MPAD // NB
    specs = ([_row_spec(a.shape[-1]) for a in ins]
             + [_full_spec(a.shape) for a in full_ins])
    out_shape = [jax.ShapeDtypeStruct((MPAD, out_cols), jnp.float32)
                 for _ in range(n_out)]
    out_specs = [_row_spec(out_cols) for _ in range(n_out)]
    if n_out == 1:
        out_shape, out_specs = out_shape[0], out_specs[0]
    return pl.pallas_call(
        body,
        grid=grid,
        in_specs=specs,
        out_specs=out_specs,
        out_shape=out_shape,
    )(*ins, *full_ins)


def _small_call(actc, d0w, d0b, actors, wc):
    return pl.pallas_call(
        _small_body,
        in_specs=[_full_spec(a.shape) for a in (actc, d0w, d0b, actors, wc)],
        out_specs=[_full_spec((N_ACT, N_D))] * 2,
        out_shape=[jax.ShapeDtypeStruct((N_ACT, N_D), jnp.float32)] * 2,
    )(actc, d0w, d0b, actors, wc)


def kernel(feat, graph_turn, graph_control, graph_intersect, graph_idcs,
           graph_ctrs, actors, actor_idcs, actor_ctrs, params):
    row = lambda a: a[None, :]
    xin = jnp.concatenate(
        [feat, graph_turn, graph_control[:, None], graph_intersect[:, None]],
        axis=1)
    xin = jnp.pad(xin, ((0, MPAD - N_MAP), (0, 256 - (N_D + 4))))
    mw = jnp.pad(params['meta_W'].T, ((0, 256 - (N_D + 4)), (0, 0)))
    ctrs = jnp.pad(graph_ctrs, ((0, MPAD - N_MAP), (0, 0)),
                   constant_values=1e9)

    x = _row_call(_pre_body, [xin],
                  [mw, row(params['meta_gw']), row(params['meta_gb'])], 1)

    for name in ('att0', 'att1'):
        p = params[name]
        w0 = p['ctx0_W']
        wd = w0[:, :N_D].T
        wq = w0[:, N_D:2 * N_D].T
        wc = w0[:, 2 * N_D:].T

        q2, acc0, g0 = _row_call(
            _layer_pre_body, [x, ctrs],
            [p['query_W'].T, row(p['query_gw']), row(p['query_gb']),
             wq, p['agt_W'].T, p['dist0_W'].T], 3)

        a0b, c2 = _small_call(actor_ctrs, p['dist0_W'].T,
                              row(p['dist0_b']), actors, wc)

        acc = _row_call(
            _att_body, [q2, g0, acc0, ctrs],
            [actor_ctrs, a0b, c2, p['dist1_W'].T, row(p['dist1_gw']),
             row(p['dist1_gb']), wd, row(p['ctx0_gw']), row(p['ctx0_gb']),
             p['ctx1_W'].T], 1)

        x = _row_call(
            _post_body, [acc, x],
            [row(p['norm_gw']), row(p['norm_gb']), p['linear_W'].T,
             row(p['linear_gw']), row(p['linear_gb'])], 1)

    return x[:N_MAP]
